# baseline (speedup 1.0000x reference)
import functools

import jax
import jax.numpy as jnp
from jax import lax
from jax.experimental import pallas as pl
from jax.experimental.pallas import tpu as pltpu
from jax.experimental.pallas import tpu_sc as plsc

_NC = 2
_NS = 16
_NW = _NC * _NS
_CHR = 256
_LANES = 16


def _mesh():
    return plsc.VectorSubcoreMesh(core_axis_name="c", subcore_axis_name="s")


def _splat(v):
    return jnp.full((_LANES,), v, jnp.int32)


@functools.lru_cache(maxsize=None)
def _build_pack(vocab, d):
    n_full = vocab // _CHR
    tail_rows = vocab - n_full * _CHR
    n_lines = vocab // 2

    @functools.partial(
        pl.kernel,
        mesh=_mesh(),
        out_type=jax.ShapeDtypeStruct((n_lines, 2 * d), jnp.float32),
        scratch_types=[
            pltpu.VMEM((2, d, _CHR + 1), jnp.float32),
            pltpu.VMEM((2, _CHR // 2, 2 * d), jnp.float32),
            pltpu.VMEM((tail_rows, 2 * d), jnp.float32),
            pltpu.SemaphoreType.DMA((2,)),
            pltpu.SemaphoreType.DMA((2,)),
        ],
        compiler_params=pltpu.CompilerParams(
            use_tc_tiling_on_sc=True, needs_layout_passes=False),
    )
    def pack_kernel(tbl_t, tail, scr, srcv, linev, tailv, isem, osem):
        wid = lax.axis_index("s") * _NC + lax.axis_index("c")
        count = 122 + jnp.where(wid < n_full - 122 * _NW, 1, 0)

        def in_copy(k, b):
            c = wid + k * _NW
            return pltpu.make_async_copy(
                tbl_t.at[:, pl.ds(c * _CHR, _CHR)],
                srcv.at[b, :, pl.ds(0, _CHR)], isem.at[b])

        def out_copy(k, b):
            c = wid + k * _NW
            return pltpu.make_async_copy(
                linev.at[b], scr.at[pl.ds(c * (_CHR // 2), _CHR // 2), :],
                osem.at[b])

        iota = lax.iota(jnp.int32, _LANES)
        rows = [iota + t * _LANES for t in range(d // _LANES)]

        def build(b):
            def body(j, carry):
                col0 = _splat(2 * j)
                col1 = col0 + 1
                for g in range(2 * d // _LANES):
                    col = col1 if g >= d // _LANES else col0
                    vals = plsc.load_gather(srcv.at[b], [rows[g % (d // _LANES)], col])
                    linev[b, j, pl.ds(g * _LANES, _LANES)] = vals
                return carry
            lax.fori_loop(0, _CHR // 2, body, 0)

        in_copy(0, 0).start()

        def pair(p, carry):
            for s in range(2):
                k = p * 2 + s
                b = s

                @pl.when(k < count)
                def _():
                    in_copy(k, b).wait()

                    @pl.when(k + 1 < count)
                    def _():
                        in_copy(k + 1, 1 - b).start()

                    build(b)

                    @pl.when(k >= 2)
                    def _():
                        out_copy(k - 2, b).wait()

                    out_copy(k, b).start()
            return carry

        lax.fori_loop(0, 62, pair, 0)

        @pl.when(count == 123)
        def _():
            out_copy(121, 1).wait()
            out_copy(122, 0).wait()

        @pl.when(count == 122)
        def _():
            out_copy(120, 0).wait()
            out_copy(121, 1).wait()

        @pl.when(wid == 0)
        def _():
            pltpu.sync_copy(tail, tailv)

            def body(j, carry):
                for g in range(2 * d // _LANES):
                    row = _splat(2 * j + (1 if g >= d // _LANES else 0))
                    col = iota + (g % (d // _LANES)) * _LANES
                    vals = plsc.load_gather(tailv, [row, col])
                    linev[0, j, pl.ds(g * _LANES, _LANES)] = vals
                return carry
            lax.fori_loop(0, tail_rows // 2, body, 0)
            pltpu.sync_copy(
                linev.at[0, pl.ds(0, tail_rows // 2)],
                scr.at[pl.ds(n_full * (_CHR // 2), tail_rows // 2), :])

    return pack_kernel


@functools.lru_cache(maxsize=None)
def _build_gather(batch, hist, vocab, d):
    bc = batch // _NW

    @functools.partial(
        pl.kernel,
        mesh=_mesh(),
        out_type=jax.ShapeDtypeStruct((hist, d, batch), jnp.float32),
        scratch_types=[
            pltpu.VMEM((hist, bc), jnp.int32),
            pltpu.VMEM((2, bc), jnp.int32),
            pltpu.VMEM((2, bc, 2 * d + 1), jnp.float32),
            pltpu.VMEM((2, d, bc), jnp.float32),
            pltpu.SemaphoreType.DMA((2,)),
            pltpu.SemaphoreType.DMA((2,)),
        ],
        compiler_params=pltpu.CompilerParams(
            use_tc_tiling_on_sc=True, needs_layout_passes=False),
    )
    def gather_kernel(x_t, scr, out, xv, idxv, linev, obuf, gsem, osem):
        wid = lax.axis_index("s") * _NC + lax.axis_index("c")
        b0 = wid * bc
        pltpu.sync_copy(x_t.at[:, pl.ds(b0, bc)], xv)
        iota = lax.iota(jnp.int32, _LANES)

        def prep(h, b):
            for g in range(bc // _LANES):
                v = xv[h, pl.ds(g * _LANES, _LANES)]
                idxv[b, pl.ds(g * _LANES, _LANES)] = (
                    lax.shift_right_logical(v, 1))

        def start_gather(b):
            pltpu.async_copy(
                scr.at[idxv.at[b]], linev.at[b, :, pl.ds(0, 2 * d)],
                gsem.at[b])

        def wait_gather(b):
            pltpu.make_async_copy(
                scr.at[idxv.at[b]], linev.at[b, :, pl.ds(0, 2 * d)],
                gsem.at[b]).wait()

        def out_copy(h, b):
            return pltpu.make_async_copy(
                obuf.at[b], out.at[h, :, pl.ds(b0, bc)], osem.at[b])

        rows = [iota + g * _LANES for g in range(bc // _LANES)]

        def build(h, b):
            pars = tuple(
                lax.shift_left(
                    jnp.bitwise_and(xv[h, pl.ds(g * _LANES, _LANES)], 1), 6)
                for g in range(bc // _LANES))

            def body(dd, carry):
                for g in range(bc // _LANES):
                    vals = plsc.load_gather(
                        linev.at[b], [rows[g], carry[g] + dd])
                    obuf[b, dd, pl.ds(g * _LANES, _LANES)] = vals
                return carry
            lax.fori_loop(0, d, body, pars)

        prep(0, 0)
        start_gather(0)
        prep(1, 1)

        def pair(p, carry):
            for s in range(2):
                h = p * 2 + s
                b = s
                wait_gather(b)

                @pl.when(h + 1 < hist)
                def _():
                    start_gather(1 - b)

                build(h, b)

                @pl.when(h >= 2)
                def _():
                    out_copy(h - 2, b).wait()

                out_copy(h, b).start()

                @pl.when(h + 2 < hist)
                def _():
                    prep(h + 2, b)
            return carry

        lax.fori_loop(0, hist // 2, pair, 0)
        out_copy(hist - 2, 0).wait()
        out_copy(hist - 1, 1).wait()

    return gather_kernel


def kernel(x, embedding):
    batch, hist = x.shape
    vocab, d = embedding.shape
    x_t = x.T.astype(jnp.int32)
    tbl_t = embedding.T
    n_full = vocab // _CHR
    tail_rows = vocab - n_full * _CHR
    tail = jnp.pad(embedding[n_full * _CHR:, :],
                   ((0, 0), (0, 2 * d - d)))
    scr = _build_pack(vocab, d)(tbl_t, tail)
    out_t = _build_gather(batch, hist, vocab, d)(x_t, scr)
    return out_t.transpose(2, 0, 1)


# speedup vs baseline: 1.5007x; 1.5007x over previous
import functools

import jax
import jax.numpy as jnp
from jax import lax
from jax.experimental import pallas as pl
from jax.experimental.pallas import tpu as pltpu
from jax.experimental.pallas import tpu_sc as plsc

_NC = 2
_NS = 16
_NW = _NC * _NS
_LANES = 16


@functools.lru_cache(maxsize=None)
def _build_gather(batch, hist, vocab, d):
    bc = batch // _NW
    mesh = plsc.VectorSubcoreMesh(core_axis_name="c", subcore_axis_name="s")

    @functools.partial(
        pl.kernel,
        mesh=mesh,
        out_type=jax.ShapeDtypeStruct((hist, d, batch), jnp.float32),
        scratch_types=[
            pltpu.VMEM((hist, bc), jnp.int32),
            pltpu.VMEM((2, bc), jnp.int32),
            pltpu.VMEM((2, bc, 2 * d), jnp.float32),
            pltpu.VMEM((2, d, bc), jnp.float32),
            pltpu.SemaphoreType.DMA((2,)),
            pltpu.SemaphoreType.DMA((2,)),
        ],
        compiler_params=pltpu.CompilerParams(needs_layout_passes=False),
    )
    def gather_kernel(x_t, scr, out, xv, idxv, linev, obuf, gsem, osem):
        wid = lax.axis_index("s") * _NC + lax.axis_index("c")
        b0 = wid * bc
        pltpu.sync_copy(x_t.at[:, pl.ds(b0, bc)], xv)
        iota = lax.iota(jnp.int32, _LANES)

        def prep(h, b):
            for g in range(bc // _LANES):
                v = xv[h, pl.ds(g * _LANES, _LANES)]
                idxv[b, pl.ds(g * _LANES, _LANES)] = (
                    lax.shift_right_logical(v, 1))

        def start_gather(b):
            pltpu.async_copy(scr.at[idxv.at[b]], linev.at[b], gsem.at[b])

        def wait_gather(b):
            pltpu.make_async_copy(
                scr.at[idxv.at[b]], linev.at[b], gsem.at[b]).wait()

        def out_copy(h, b):
            return pltpu.make_async_copy(
                obuf.at[b], out.at[h, :, pl.ds(b0, bc)], osem.at[b])

        rows = [iota + g * _LANES for g in range(bc // _LANES)]

        def build(h, b):
            pars = tuple(
                lax.shift_left(
                    jnp.bitwise_and(xv[h, pl.ds(g * _LANES, _LANES)], 1), 6)
                for g in range(bc // _LANES))

            def body(dd, carry):
                for g in range(bc // _LANES):
                    vals = plsc.load_gather(
                        linev.at[b], [rows[g], carry[g] + dd])
                    obuf[b, dd, pl.ds(g * _LANES, _LANES)] = vals
                return carry
            lax.fori_loop(0, d, body, pars)

        prep(0, 0)
        start_gather(0)
        prep(1, 1)

        def pair(p, carry):
            for s in range(2):
                h = p * 2 + s
                b = s
                wait_gather(b)

                @pl.when(h + 1 < hist)
                def _():
                    start_gather(1 - b)

                build(h, b)

                @pl.when(h >= 2)
                def _():
                    out_copy(h - 2, b).wait()

                out_copy(h, b).start()

                @pl.when(h + 2 < hist)
                def _():
                    prep(h + 2, b)
            return carry

        lax.fori_loop(0, hist // 2, pair, 0)
        out_copy(hist - 2, 0).wait()
        out_copy(hist - 1, 1).wait()

    return gather_kernel


def kernel(x, embedding):
    batch, hist = x.shape
    vocab, d = embedding.shape
    x_t = x.T.astype(jnp.int32)
    scr = embedding.reshape(vocab // 2, 2 * d)
    out_t = _build_gather(batch, hist, vocab, d)(x_t, scr)
    return out_t.transpose(2, 0, 1)
